# baseline (speedup 1.0000x reference)
import functools

import jax
import jax.numpy as jnp
from jax import lax
from jax.experimental import pallas as pl
from jax.experimental.pallas import tpu as pltpu
from jax.experimental.pallas import tpu_sc as plsc

NC = 2
NS = 16
K = 40


def _sc_segment_sums(e0, e1, x, ea, z128):
    N, D = x.shape
    E = e0.shape[0]
    CPT = E // (NS * K)
    NPAIRS = CPT // 2
    SLAB = 640
    STEP = 624
    assert (NS - 1) * STEP + SLAB == N and STEP % 8 == 0
    f32 = jnp.float32

    mesh = plsc.VectorSubcoreMesh(
        core_axis_name="c", subcore_axis_name="s", num_cores=NC, num_subcores=NS)

    @functools.partial(
        pl.kernel,
        out_type=[
            jax.ShapeDtypeStruct((N, D), f32),
            jax.ShapeDtypeStruct((N, D), f32),
            jax.ShapeDtypeStruct((N, D), f32),
            jax.ShapeDtypeStruct((N, D), f32),
            jax.ShapeDtypeStruct((NS, N), f32),
            jax.ShapeDtypeStruct((NS, N), f32),
        ],
        mesh=mesh,
        scratch_types=[
            pltpu.VMEM_SHARED((N, D), f32),
            [pltpu.VMEM((K,), jnp.int32) for _ in range(4)],
            [pltpu.VMEM((K,), jnp.int32) for _ in range(4)],
            pltpu.VMEM((4 * K, D), f32),
            pltpu.VMEM((N,), f32),
            pltpu.SemaphoreType.DMA,
            pltpu.SemaphoreType.DMA,
            pltpu.SemaphoreType.DMA,
        ],
        compiler_params=pltpu.CompilerParams(needs_layout_passes=False),
    )
    def sc_kernel(e0_hbm, e1_hbm, x_hbm, ea_hbm, z128_hbm,
                  xs1_out, xs2_out, ea1_out, ea2_out, dg1_out, dg2_out,
                  acc_sh, sidx, didx, rows, degp, semi, semg, sems):
        c = lax.axis_index("c")
        s = lax.axis_index("s")
        base = s * STEP
        chunkbase = s * CPT
        ones16 = jnp.ones((16,), f32)
        lane = lax.broadcasted_iota(jnp.int32, (16,), 0)

        def zero_slab():
            pltpu.sync_copy(z128_hbm.at[pl.ds(base, SLAB)],
                            acc_sh.at[pl.ds(base, SLAB)])

        def copy_out(out):
            pltpu.sync_copy(acc_sh.at[pl.ds(base, SLAB)],
                            out.at[pl.ds(base, SLAB)])

        def run_phase(mode, src_hbm, dst_hbm):
            def off(P, r):
                return (chunkbase + 2 * P + r) * K

            def idx_copies(P, sp):
                cps = []
                for r in range(2):
                    if mode == "gather":
                        cps.append((src_hbm.at[pl.ds(off(P, r), K)],
                                    sidx[2 * sp + r]))
                    cps.append((dst_hbm.at[pl.ds(off(P, r), K)],
                                didx[2 * sp + r]))
                return cps

            def rows_copies(P, sp):
                cps = []
                for r in range(2):
                    sl = rows.at[pl.ds((2 * sp + r) * K, K)]
                    if mode == "gather":
                        cps.append((x_hbm.at[sidx[2 * sp + r]], sl))
                    else:
                        cps.append((ea_hbm.at[pl.ds(off(P, r), K)], sl))
                return cps

            def scat_copies(sp):
                return [(rows.at[pl.ds((2 * sp + r) * K, K)],
                         acc_sh.at[didx[2 * sp + r]]) for r in range(2)]

            def fire(cps, sem, add=False):
                for a, b in cps:
                    pltpu.async_copy(a, b, sem, add=add)

            def drain(cps, sem):
                for a, b in cps:
                    pltpu.make_async_copy(a, b, sem).wait()

            def count_deg(sp):
                if mode != "gather":
                    return
                for r in range(2):
                    buf = didx[2 * sp + r]
                    n_full = K // 16
                    for i in range(n_full):
                        plsc.addupdate_scatter(degp, [buf[pl.ds(i * 16, 16)]],
                                               ones16)
                    rem = K - n_full * 16
                    if rem:
                        plsc.addupdate_scatter(degp,
                                               [buf[pl.ds(K - 16, 16)]],
                                               ones16, mask=lane >= 16 - rem)

            def body(P, sp, first):
                drain(rows_copies(P, sp), semg)
                fire(scat_copies(sp), sems, add=True)
                if not first:
                    drain(scat_copies(1 - sp), sems)
                fire(idx_copies(P + 1, 1 - sp), semi)
                drain(idx_copies(P + 1, 1 - sp), semi)
                count_deg(1 - sp)
                fire(rows_copies(P + 1, 1 - sp), semg)

            fire(idx_copies(0, 0), semi)
            drain(idx_copies(0, 0), semi)
            count_deg(0)
            fire(rows_copies(0, 0), semg)
            body(0, 0, first=True)

            @pl.loop(0, (NPAIRS - 2) // 2)
            def _(j2):
                P = 2 * j2 + 1
                body(P, 1, first=False)
                body(P + 1, 0, first=False)

            if NPAIRS % 2 == 1:
                body(NPAIRS - 2, 1, first=False)
                last_sp = 0
            else:
                last_sp = 1
            drain(rows_copies(NPAIRS - 1, last_sp), semg)
            fire(scat_copies(last_sp), sems, add=True)
            drain(scat_copies(1 - last_sp), sems)
            drain(scat_copies(last_sp), sems)

        def run_direction(src_hbm, dst_hbm, xs_out, ea_out, dg_out):
            @pl.loop(0, N // 16)
            def _(i):
                degp[pl.ds(i * 16, 16)] = jnp.zeros((16,), f32)

            zero_slab()
            plsc.subcore_barrier()
            run_phase("gather", src_hbm, dst_hbm)
            plsc.subcore_barrier()
            copy_out(xs_out)
            pltpu.sync_copy(degp, dg_out.at[s])
            plsc.subcore_barrier()
            zero_slab()
            plsc.subcore_barrier()
            run_phase("seq", src_hbm, dst_hbm)
            plsc.subcore_barrier()
            copy_out(ea_out)

        @pl.when(c == 0)
        def _():
            run_direction(e0_hbm, e1_hbm, xs1_out, ea1_out, dg1_out)

        @pl.when(c == 1)
        def _():
            run_direction(e1_hbm, e0_hbm, xs2_out, ea2_out, dg2_out)

    return sc_kernel(e0, e1, x, ea, z128)


def _combine_body(x_ref, xs1_ref, xs2_ref, ea1_ref, ea2_ref, d1_ref, d2_ref,
                  wn1_ref, we1_ref, wm1_ref, bm1_ref,
                  wn2_ref, we2_ref, wm2_ref, bm2_ref, o_ref):
    f32 = jnp.float32
    D = x_ref.shape[1]

    def mm_t(a, b):
        return lax.dot_general(a, b, (((1,), (1,)), ((), ())),
                               preferred_element_type=f32)

    xb = x_ref[...]
    wn1 = wn1_ref[...]
    wn2 = wn2_ref[...]
    wm1 = wm1_ref[...]
    wm2 = wm2_ref[...]
    d1 = jnp.sum(d1_ref[...], axis=1, keepdims=True)
    d2 = jnp.sum(d2_ref[...], axis=1, keepdims=True)
    agg = (mm_t(mm_t(xs1_ref[...], wn1), wm1[:, 0:D])
           + d1 * mm_t(mm_t(xb, wn1), wm1[:, D:2 * D])
           + mm_t(mm_t(ea1_ref[...], we1_ref[...]), wm1[:, 2 * D:3 * D])
           + d1 * bm1_ref[...]
           + mm_t(mm_t(xs2_ref[...], wn2), wm2[:, 0:D])
           + d2 * mm_t(mm_t(xb, wn2), wm2[:, D:2 * D])
           + mm_t(mm_t(ea2_ref[...], we2_ref[...]), wm2[:, 2 * D:3 * D])
           + d2 * bm2_ref[...])
    o_ref[...] = xb + 0.5 * agg


def _combine(x, xs1, xs2, ea1, ea2, dg1, dg2,
             Wn1, We1, Wm1, bm1, Wn2, We2, Wm2, bm2):
    N, D = x.shape
    NB = 1000
    grid = (N // NB,)
    row_spec = pl.BlockSpec((NB, D), lambda i: (i, 0))
    deg_spec = pl.BlockSpec((NB, NS), lambda i: (i, 0))
    w_spec = pl.BlockSpec((D, D), lambda i: (0, 0))
    wm_spec = pl.BlockSpec((D, 3 * D), lambda i: (0, 0))
    b_spec = pl.BlockSpec((1, D), lambda i: (0, 0))
    return pl.pallas_call(
        _combine_body,
        grid=grid,
        in_specs=[row_spec, row_spec, row_spec, row_spec, row_spec,
                  deg_spec, deg_spec,
                  w_spec, w_spec, wm_spec, b_spec,
                  w_spec, w_spec, wm_spec, b_spec],
        out_specs=row_spec,
        out_shape=jax.ShapeDtypeStruct((N, D), jnp.float32),
    )(x, xs1, xs2, ea1, ea2, dg1, dg2,
      Wn1, We1, Wm1, bm1, Wn2, We2, Wm2, bm2)


def kernel(x, edge_index, edge_attr, memory, batch_id,
           Wn_s2t, We_s2t, Wm_s2t, bm_s2t,
           Wn_t2s, We_t2s, Wm_t2s, bm_t2s):
    N, D = x.shape
    E = edge_index.shape[1]
    assert E % (NS * K * 2) == 0 and N % 1000 == 0 and N % 16 == 0

    z128 = jnp.zeros((N, D), jnp.float32)

    xs1, xs2, ea1, ea2, dg1, dg2 = _sc_segment_sums(
        edge_index[0], edge_index[1], x, edge_attr, z128)
    dg1 = dg1.T
    dg2 = dg2.T

    out = _combine(x, xs1, xs2, ea1, ea2, dg1, dg2,
                   Wn_s2t, We_s2t, Wm_s2t, jnp.reshape(bm_s2t, (1, D)),
                   Wn_t2s, We_t2s, Wm_t2s, jnp.reshape(bm_t2s, (1, D)))
    return (out, edge_attr)


# speedup vs baseline: 1.0110x; 1.0110x over previous
import functools

import jax
import jax.numpy as jnp
from jax import lax
from jax.experimental import pallas as pl
from jax.experimental.pallas import tpu as pltpu
from jax.experimental.pallas import tpu_sc as plsc

NC = 2
NS = 16
K = 80


def _sc_segment_sums(e0, e1, x, ea, z128):
    N, D = x.shape
    E = e0.shape[0]
    CPT = E // (NS * K)
    SLAB = 640
    STEP = 624
    assert (NS - 1) * STEP + SLAB == N and STEP % 8 == 0
    f32 = jnp.float32

    mesh = plsc.VectorSubcoreMesh(
        core_axis_name="c", subcore_axis_name="s", num_cores=NC, num_subcores=NS)

    @functools.partial(
        pl.kernel,
        out_type=[
            jax.ShapeDtypeStruct((N, D), f32),
            jax.ShapeDtypeStruct((N, D), f32),
            jax.ShapeDtypeStruct((N, D), f32),
            jax.ShapeDtypeStruct((N, D), f32),
            jax.ShapeDtypeStruct((NS, N), f32),
            jax.ShapeDtypeStruct((NS, N), f32),
        ],
        mesh=mesh,
        scratch_types=[
            pltpu.VMEM_SHARED((N, D), f32),
            [pltpu.VMEM((K,), jnp.int32) for _ in range(2)],
            [pltpu.VMEM((K,), jnp.int32) for _ in range(2)],
            pltpu.VMEM((2 * K, D), f32),
            pltpu.VMEM((N,), f32),
            pltpu.SemaphoreType.DMA,
            pltpu.SemaphoreType.DMA,
            pltpu.SemaphoreType.DMA,
        ],
        compiler_params=pltpu.CompilerParams(needs_layout_passes=False),
    )
    def sc_kernel(e0_hbm, e1_hbm, x_hbm, ea_hbm, z128_hbm,
                  xs1_out, xs2_out, ea1_out, ea2_out, dg1_out, dg2_out,
                  acc_sh, sidx, didx, rows, degp, semi, semg, sems):
        c = lax.axis_index("c")
        s = lax.axis_index("s")
        base = s * STEP
        chunkbase = s * CPT
        ones16 = jnp.ones((16,), f32)
        lane = lax.broadcasted_iota(jnp.int32, (16,), 0)

        def zero_slab():
            pltpu.sync_copy(z128_hbm.at[pl.ds(base, SLAB)],
                            acc_sh.at[pl.ds(base, SLAB)])

        def copy_out(out):
            pltpu.sync_copy(acc_sh.at[pl.ds(base, SLAB)],
                            out.at[pl.ds(base, SLAB)])

        def run_phase(mode, src_hbm, dst_hbm):
            def off(P):
                return (chunkbase + P) * K

            def idx_copies(P, sp):
                cps = []
                if mode == "gather":
                    cps.append((src_hbm.at[pl.ds(off(P), K)], sidx[sp]))
                cps.append((dst_hbm.at[pl.ds(off(P), K)], didx[sp]))
                return cps

            def rows_copies(P, sp):
                sl = rows.at[pl.ds(sp * K, K)]
                if mode == "gather":
                    return [(x_hbm.at[sidx[sp]], sl)]
                return [(ea_hbm.at[pl.ds(off(P), K)], sl)]

            def scat_copies(sp):
                return [(rows.at[pl.ds(sp * K, K)], acc_sh.at[didx[sp]])]

            def fire(cps, sem, add=False):
                for a, b in cps:
                    pltpu.async_copy(a, b, sem, add=add)

            def drain(cps, sem):
                for a, b in cps:
                    pltpu.make_async_copy(a, b, sem).wait()

            def count_deg(sp):
                if mode != "gather":
                    return
                buf = didx[sp]
                for i in range(K // 16):
                    plsc.addupdate_scatter(degp, [buf[pl.ds(i * 16, 16)]],
                                           ones16)

            def body(P, sp, first):
                drain(rows_copies(P, sp), semg)
                fire(scat_copies(sp), sems, add=True)
                if not first:
                    drain(scat_copies(1 - sp), sems)
                fire(idx_copies(P + 1, 1 - sp), semi)
                drain(idx_copies(P + 1, 1 - sp), semi)
                count_deg(1 - sp)
                fire(rows_copies(P + 1, 1 - sp), semg)

            fire(idx_copies(0, 0), semi)
            drain(idx_copies(0, 0), semi)
            count_deg(0)
            fire(rows_copies(0, 0), semg)
            body(0, 0, first=True)

            @pl.loop(0, (CPT - 2) // 2)
            def _(j2):
                P = 2 * j2 + 1
                body(P, 1, first=False)
                body(P + 1, 0, first=False)

            if CPT % 2 == 1:
                body(CPT - 2, 1, first=False)
                last_sp = 0
            else:
                last_sp = 1
            drain(rows_copies(CPT - 1, last_sp), semg)
            fire(scat_copies(last_sp), sems, add=True)
            drain(scat_copies(1 - last_sp), sems)
            drain(scat_copies(last_sp), sems)

        def run_direction(src_hbm, dst_hbm, xs_out, ea_out, dg_out):
            @pl.loop(0, N // 16)
            def _(i):
                degp[pl.ds(i * 16, 16)] = jnp.zeros((16,), f32)

            zero_slab()
            plsc.subcore_barrier()
            run_phase("gather", src_hbm, dst_hbm)
            plsc.subcore_barrier()
            copy_out(xs_out)
            pltpu.sync_copy(degp, dg_out.at[s])
            plsc.subcore_barrier()
            zero_slab()
            plsc.subcore_barrier()
            run_phase("seq", src_hbm, dst_hbm)
            plsc.subcore_barrier()
            copy_out(ea_out)

        @pl.when(c == 0)
        def _():
            run_direction(e0_hbm, e1_hbm, xs1_out, ea1_out, dg1_out)

        @pl.when(c == 1)
        def _():
            run_direction(e1_hbm, e0_hbm, xs2_out, ea2_out, dg2_out)

    return sc_kernel(e0, e1, x, ea, z128)


def _combine_body(x_ref, xs1_ref, xs2_ref, ea1_ref, ea2_ref, d1_ref, d2_ref,
                  wn1_ref, we1_ref, wm1_ref, bm1_ref,
                  wn2_ref, we2_ref, wm2_ref, bm2_ref, o_ref):
    f32 = jnp.float32
    D = x_ref.shape[1]

    def mm_t(a, b):
        return lax.dot_general(a, b, (((1,), (1,)), ((), ())),
                               preferred_element_type=f32)

    xb = x_ref[...]
    wn1 = wn1_ref[...]
    wn2 = wn2_ref[...]
    wm1 = wm1_ref[...]
    wm2 = wm2_ref[...]
    d1 = jnp.sum(d1_ref[...], axis=1, keepdims=True)
    d2 = jnp.sum(d2_ref[...], axis=1, keepdims=True)
    agg = (mm_t(mm_t(xs1_ref[...], wn1), wm1[:, 0:D])
           + d1 * mm_t(mm_t(xb, wn1), wm1[:, D:2 * D])
           + mm_t(mm_t(ea1_ref[...], we1_ref[...]), wm1[:, 2 * D:3 * D])
           + d1 * bm1_ref[...]
           + mm_t(mm_t(xs2_ref[...], wn2), wm2[:, 0:D])
           + d2 * mm_t(mm_t(xb, wn2), wm2[:, D:2 * D])
           + mm_t(mm_t(ea2_ref[...], we2_ref[...]), wm2[:, 2 * D:3 * D])
           + d2 * bm2_ref[...])
    o_ref[...] = xb + 0.5 * agg


def _combine(x, xs1, xs2, ea1, ea2, dg1, dg2,
             Wn1, We1, Wm1, bm1, Wn2, We2, Wm2, bm2):
    N, D = x.shape
    NB = 1000
    grid = (N // NB,)
    row_spec = pl.BlockSpec((NB, D), lambda i: (i, 0))
    deg_spec = pl.BlockSpec((NB, NS), lambda i: (i, 0))
    w_spec = pl.BlockSpec((D, D), lambda i: (0, 0))
    wm_spec = pl.BlockSpec((D, 3 * D), lambda i: (0, 0))
    b_spec = pl.BlockSpec((1, D), lambda i: (0, 0))
    return pl.pallas_call(
        _combine_body,
        grid=grid,
        in_specs=[row_spec, row_spec, row_spec, row_spec, row_spec,
                  deg_spec, deg_spec,
                  w_spec, w_spec, wm_spec, b_spec,
                  w_spec, w_spec, wm_spec, b_spec],
        out_specs=row_spec,
        out_shape=jax.ShapeDtypeStruct((N, D), jnp.float32),
    )(x, xs1, xs2, ea1, ea2, dg1, dg2,
      Wn1, We1, Wm1, bm1, Wn2, We2, Wm2, bm2)


def kernel(x, edge_index, edge_attr, memory, batch_id,
           Wn_s2t, We_s2t, Wm_s2t, bm_s2t,
           Wn_t2s, We_t2s, Wm_t2s, bm_t2s):
    N, D = x.shape
    E = edge_index.shape[1]
    assert E % (NS * K * 2) == 0 and N % 1000 == 0 and N % 16 == 0

    z128 = jnp.zeros((N, D), jnp.float32)

    xs1, xs2, ea1, ea2, dg1, dg2 = _sc_segment_sums(
        edge_index[0], edge_index[1], x, edge_attr, z128)
    dg1 = dg1.T
    dg2 = dg2.T

    out = _combine(x, xs1, xs2, ea1, ea2, dg1, dg2,
                   Wn_s2t, We_s2t, Wm_s2t, jnp.reshape(bm_s2t, (1, D)),
                   Wn_t2s, We_t2s, Wm_t2s, jnp.reshape(bm_t2s, (1, D)))
    return (out, edge_attr)


# speedup vs baseline: 1.1862x; 1.1733x over previous
import functools

import jax
import jax.numpy as jnp
from jax import lax
from jax.experimental import pallas as pl
from jax.experimental.pallas import tpu as pltpu
from jax.experimental.pallas import tpu_sc as plsc

NC = 2
NS = 16
K = 80


def _sc_segment_sums(e0, e1, x, ea, z128):
    N, D = x.shape
    E = e0.shape[0]
    CPT = E // (NS * K)
    SLAB = 640
    STEP = 624
    assert (NS - 1) * STEP + SLAB == N and STEP % 8 == 0
    f32 = jnp.float32

    mesh = plsc.VectorSubcoreMesh(
        core_axis_name="c", subcore_axis_name="s", num_cores=NC, num_subcores=NS)

    @functools.partial(
        pl.kernel,
        out_type=[
            jax.ShapeDtypeStruct((N, D), f32),
            jax.ShapeDtypeStruct((N, D), f32),
            jax.ShapeDtypeStruct((N, D), f32),
            jax.ShapeDtypeStruct((N, D), f32),
            jax.ShapeDtypeStruct((NS, K, D), f32),
            jax.ShapeDtypeStruct((NS, K, D), f32),
        ],
        mesh=mesh,
        scratch_types=[
            pltpu.VMEM_SHARED((N, D), f32),
            [pltpu.VMEM((K,), jnp.int32) for _ in range(4)],
            [pltpu.VMEM((K,), jnp.int32) for _ in range(4)],
            pltpu.VMEM((4 * K, D), f32),
            pltpu.SemaphoreType.DMA,
            pltpu.SemaphoreType.DMA,
            pltpu.SemaphoreType.DMA,
        ],
        compiler_params=pltpu.CompilerParams(needs_layout_passes=False),
    )
    def sc_kernel(e0_hbm, e1_hbm, x_hbm, ea_hbm, z128_hbm,
                  xs1_out, xs2_out, ea1_out, ea2_out, dg1_out, dg2_out,
                  acc_sh, sidx, didx, rows, semi, semg, sems):
        c = lax.axis_index("c")
        s = lax.axis_index("s")
        base = s * STEP
        chunkbase = s * CPT
        NPAIRS = CPT // 2
        ones16 = jnp.ones((16,), f32)

        def zero_slab():
            pltpu.sync_copy(z128_hbm.at[pl.ds(base, SLAB)],
                            acc_sh.at[pl.ds(base, SLAB)])

        def copy_out(out):
            pltpu.sync_copy(acc_sh.at[pl.ds(base, SLAB)],
                            out.at[pl.ds(base, SLAB)])

        def run_phase(mode, src_hbm, dst_hbm):
            def off(P, r):
                return (chunkbase + 2 * P + r) * K

            def idx_copies(P, sp):
                cps = []
                for r in range(2):
                    if mode == "gather":
                        cps.append((src_hbm.at[pl.ds(off(P, r), K)],
                                    sidx[2 * sp + r]))
                    cps.append((dst_hbm.at[pl.ds(off(P, r), K)],
                                didx[2 * sp + r]))
                return cps

            def rows_copies(P, sp):
                cps = []
                for r in range(2):
                    sl = rows.at[pl.ds((2 * sp + r) * K, K)]
                    if mode == "gather":
                        cps.append((x_hbm.at[sidx[2 * sp + r]], sl))
                    else:
                        cps.append((ea_hbm.at[pl.ds(off(P, r), K)], sl))
                return cps

            def scat_copies(sp):
                return [(rows.at[pl.ds((2 * sp + r) * K, K)],
                         acc_sh.at[didx[2 * sp + r]]) for r in range(2)]

            def fire(cps, sem, add=False):
                for a, b in cps:
                    pltpu.async_copy(a, b, sem, add=add)

            def drain(cps, sem):
                for a, b in cps:
                    pltpu.make_async_copy(a, b, sem).wait()

            def body(P, sp, first):
                drain(rows_copies(P, sp), semg)
                fire(scat_copies(sp), sems, add=True)
                if not first:
                    drain(scat_copies(1 - sp), sems)
                fire(idx_copies(P + 1, 1 - sp), semi)
                drain(idx_copies(P + 1, 1 - sp), semi)
                fire(rows_copies(P + 1, 1 - sp), semg)

            fire(idx_copies(0, 0), semi)
            drain(idx_copies(0, 0), semi)
            fire(rows_copies(0, 0), semg)
            body(0, 0, first=True)

            @pl.loop(0, (NPAIRS - 2) // 2)
            def _(j2):
                P = 2 * j2 + 1
                body(P, 1, first=False)
                body(P + 1, 0, first=False)

            if NPAIRS % 2 == 1:
                body(NPAIRS - 2, 1, first=False)
                last_sp = 0
            else:
                last_sp = 1
            drain(rows_copies(NPAIRS - 1, last_sp), semg)
            fire(scat_copies(last_sp), sems, add=True)
            drain(scat_copies(1 - last_sp), sems)
            drain(scat_copies(last_sp), sems)

        def run_count(dst_hbm):
            def didx_cp(P, sp):
                cn = jnp.minimum(P, CPT - 1)
                return [(dst_hbm.at[pl.ds((chunkbase + cn) * K, K)], didx[sp])]

            def count(sp):
                buf = didx[sp]
                for i in range(K // 16):
                    idx16 = buf[pl.ds(i * 16, 16)]
                    plsc.addupdate_scatter(
                        rows, [lax.shift_right_logical(idx16, 7),
                               lax.bitwise_and(idx16, 127)], ones16)

            def step(P, sp):
                for a, b in didx_cp(P + 1, 1 - sp):
                    pltpu.async_copy(a, b, semi)
                count(sp)
                for a, b in didx_cp(P + 1, 1 - sp):
                    pltpu.make_async_copy(a, b, semi).wait()

            pltpu.sync_copy(z128_hbm.at[pl.ds(0, K)], rows.at[pl.ds(0, K)])
            for a, b in didx_cp(0, 0):
                pltpu.async_copy(a, b, semi)
                pltpu.make_async_copy(a, b, semi).wait()
            step(0, 0)

            @pl.loop(0, (CPT - 2) // 2)
            def _(j2):
                P = 2 * j2 + 1
                step(P, 1)
                step(P + 1, 0)

            count(1 if CPT % 2 == 0 else 0)

        def run_direction(src_hbm, dst_hbm, xs_out, ea_out, dg_out):
            zero_slab()
            plsc.subcore_barrier()
            run_phase("gather", src_hbm, dst_hbm)
            plsc.subcore_barrier()
            copy_out(xs_out)
            plsc.subcore_barrier()
            zero_slab()
            plsc.subcore_barrier()
            run_phase("seq", src_hbm, dst_hbm)
            run_count(dst_hbm)
            plsc.subcore_barrier()
            copy_out(ea_out)
            pltpu.sync_copy(rows.at[pl.ds(0, K)], dg_out.at[s])

        @pl.when(c == 0)
        def _():
            run_direction(e0_hbm, e1_hbm, xs1_out, ea1_out, dg1_out)

        @pl.when(c == 1)
        def _():
            run_direction(e1_hbm, e0_hbm, xs2_out, ea2_out, dg2_out)

    return sc_kernel(e0, e1, x, ea, z128)


def _combine_body(x_ref, xs1_ref, xs2_ref, ea1_ref, ea2_ref, d1_ref, d2_ref,
                  wn1_ref, we1_ref, wm1_ref, bm1_ref,
                  wn2_ref, we2_ref, wm2_ref, bm2_ref, o_ref):
    f32 = jnp.float32
    D = x_ref.shape[1]

    def mm_t(a, b):
        return lax.dot_general(a, b, (((1,), (1,)), ((), ())),
                               preferred_element_type=f32)

    xb = x_ref[...]
    wn1 = wn1_ref[...]
    wn2 = wn2_ref[...]
    wm1 = wm1_ref[...]
    wm2 = wm2_ref[...]
    d1 = jnp.sum(d1_ref[...], axis=1, keepdims=True)
    d2 = jnp.sum(d2_ref[...], axis=1, keepdims=True)
    agg = (mm_t(mm_t(xs1_ref[...], wn1), wm1[:, 0:D])
           + d1 * mm_t(mm_t(xb, wn1), wm1[:, D:2 * D])
           + mm_t(mm_t(ea1_ref[...], we1_ref[...]), wm1[:, 2 * D:3 * D])
           + d1 * bm1_ref[...]
           + mm_t(mm_t(xs2_ref[...], wn2), wm2[:, 0:D])
           + d2 * mm_t(mm_t(xb, wn2), wm2[:, D:2 * D])
           + mm_t(mm_t(ea2_ref[...], we2_ref[...]), wm2[:, 2 * D:3 * D])
           + d2 * bm2_ref[...])
    o_ref[...] = xb + 0.5 * agg


def _combine(x, xs1, xs2, ea1, ea2, dg1, dg2,
             Wn1, We1, Wm1, bm1, Wn2, We2, Wm2, bm2):
    N, D = x.shape
    NB = 1000
    grid = (N // NB,)
    row_spec = pl.BlockSpec((NB, D), lambda i: (i, 0))
    deg_spec = pl.BlockSpec((NB, NS), lambda i: (i, 0))
    w_spec = pl.BlockSpec((D, D), lambda i: (0, 0))
    wm_spec = pl.BlockSpec((D, 3 * D), lambda i: (0, 0))
    b_spec = pl.BlockSpec((1, D), lambda i: (0, 0))
    return pl.pallas_call(
        _combine_body,
        grid=grid,
        in_specs=[row_spec, row_spec, row_spec, row_spec, row_spec,
                  deg_spec, deg_spec,
                  w_spec, w_spec, wm_spec, b_spec,
                  w_spec, w_spec, wm_spec, b_spec],
        out_specs=row_spec,
        out_shape=jax.ShapeDtypeStruct((N, D), jnp.float32),
    )(x, xs1, xs2, ea1, ea2, dg1, dg2,
      Wn1, We1, Wm1, bm1, Wn2, We2, Wm2, bm2)


def kernel(x, edge_index, edge_attr, memory, batch_id,
           Wn_s2t, We_s2t, Wm_s2t, bm_s2t,
           Wn_t2s, We_t2s, Wm_t2s, bm_t2s):
    N, D = x.shape
    E = edge_index.shape[1]
    assert E % (NS * K * 2) == 0 and N % 1000 == 0 and N % 16 == 0

    z128 = jnp.zeros((N, D), jnp.float32)

    xs1, xs2, ea1, ea2, dg1, dg2 = _sc_segment_sums(
        edge_index[0], edge_index[1], x, edge_attr, z128)
    assert K * D >= N
    dg1 = dg1.reshape(NS, K * D)[:, :N].T
    dg2 = dg2.reshape(NS, K * D)[:, :N].T

    out = _combine(x, xs1, xs2, ea1, ea2, dg1, dg2,
                   Wn_s2t, We_s2t, Wm_s2t, jnp.reshape(bm_s2t, (1, D)),
                   Wn_t2s, We_t2s, Wm_t2s, jnp.reshape(bm_t2s, (1, D)))
    return (out, edge_attr)


# speedup vs baseline: 1.3598x; 1.1463x over previous
import functools

import jax
import jax.numpy as jnp
from jax import lax
from jax.experimental import pallas as pl
from jax.experimental.pallas import tpu as pltpu
from jax.experimental.pallas import tpu_sc as plsc

NC = 2
NS = 16
K = 80


def _sc_segment_sums(e0, e1, x, ea, z128):
    N, D = x.shape
    E = e0.shape[0]
    CPT = E // (NS * K)
    SLAB = 640
    STEP = 624
    assert (NS - 1) * STEP + SLAB == N and STEP % 8 == 0
    f32 = jnp.float32

    mesh = plsc.VectorSubcoreMesh(
        core_axis_name="c", subcore_axis_name="s", num_cores=NC, num_subcores=NS)

    @functools.partial(
        pl.kernel,
        out_type=[
            jax.ShapeDtypeStruct((N, D), f32),
            jax.ShapeDtypeStruct((N, D), f32),
            jax.ShapeDtypeStruct((N, D), f32),
            jax.ShapeDtypeStruct((N, D), f32),
            jax.ShapeDtypeStruct((NS, K, D), f32),
            jax.ShapeDtypeStruct((NS, K, D), f32),
        ],
        mesh=mesh,
        scratch_types=[
            pltpu.VMEM_SHARED((N, D), f32),
            [pltpu.VMEM((K,), jnp.int32) for _ in range(8)],
            [pltpu.VMEM((K,), jnp.int32) for _ in range(8)],
            pltpu.VMEM((4 * K, D), f32),
            pltpu.SemaphoreType.DMA,
            pltpu.SemaphoreType.DMA,
            pltpu.SemaphoreType.DMA,
        ],
        compiler_params=pltpu.CompilerParams(needs_layout_passes=False),
    )
    def sc_kernel(e0_hbm, e1_hbm, x_hbm, ea_hbm, z128_hbm,
                  xs1_out, xs2_out, ea1_out, ea2_out, dg1_out, dg2_out,
                  acc_sh, sidx, didx, rows, semi, semg, sems):
        c = lax.axis_index("c")
        s = lax.axis_index("s")
        base = s * STEP
        chunkbase = s * CPT
        NPAIRS = CPT // 2
        ones16 = jnp.ones((16,), f32)

        def zero_slab():
            pltpu.sync_copy(z128_hbm.at[pl.ds(base, SLAB)],
                            acc_sh.at[pl.ds(base, SLAB)])

        def copy_out(out):
            pltpu.sync_copy(acc_sh.at[pl.ds(base, SLAB)],
                            out.at[pl.ds(base, SLAB)])

        def run_phase(mode, src_hbm, dst_hbm):
            def off(P, r):
                Pc = jnp.minimum(P, NPAIRS - 1)
                return (chunkbase + 2 * Pc + r) * K

            def idx_copies(P, ip):
                cps = []
                for r in range(2):
                    if mode == "gather":
                        cps.append((src_hbm.at[pl.ds(off(P, r), K)],
                                    sidx[2 * ip + r]))
                    cps.append((dst_hbm.at[pl.ds(off(P, r), K)],
                                didx[2 * ip + r]))
                return cps

            def rows_copies(P, sp, ip):
                cps = []
                for r in range(2):
                    sl = rows.at[pl.ds((2 * sp + r) * K, K)]
                    if mode == "gather":
                        cps.append((x_hbm.at[sidx[2 * ip + r]], sl))
                    else:
                        cps.append((ea_hbm.at[pl.ds(off(P, r), K)], sl))
                return cps

            def scat_copies(sp, ip):
                return [(rows.at[pl.ds((2 * sp + r) * K, K)],
                         acc_sh.at[didx[2 * ip + r]]) for r in range(2)]

            def fire(cps, sem, add=False):
                for a, b in cps:
                    pltpu.async_copy(a, b, sem, add=add)

            def drain(cps, sem):
                for a, b in cps:
                    pltpu.make_async_copy(a, b, sem).wait()

            def body(P, su, first):
                sp = su % 2
                ip = su % 4
                drain(idx_copies(P + 1, (su + 1) % 4), semi)
                drain(rows_copies(P, sp, ip), semg)
                fire(scat_copies(sp, ip), sems, add=True)
                if not first:
                    drain(scat_copies(1 - sp, (su - 1) % 4), sems)
                fire(rows_copies(P + 1, 1 - sp, (su + 1) % 4), semg)
                fire(idx_copies(P + 2, (su + 2) % 4), semi)

            fire(idx_copies(0, 0), semi)
            drain(idx_copies(0, 0), semi)
            fire(rows_copies(0, 0, 0), semg)
            fire(idx_copies(1, 1), semi)
            body(0, 0, first=True)

            @pl.loop(0, (NPAIRS - 1) // 4)
            def _(j4):
                for u in range(4):
                    body(4 * j4 + 1 + u, 1 + u, first=False)

            last = NPAIRS - 1
            drain(scat_copies(last % 2, last % 4), sems)
            drain(rows_copies(last + 1, (last + 1) % 2, (last + 1) % 4), semg)
            drain(idx_copies(last + 2, (last + 2) % 4), semi)

        def run_count(dst_hbm):
            def didx_cp(P, sp):
                cn = jnp.minimum(P, CPT - 1)
                return [(dst_hbm.at[pl.ds((chunkbase + cn) * K, K)], didx[sp])]

            def count(sp):
                buf = didx[sp]
                for i in range(K // 16):
                    idx16 = buf[pl.ds(i * 16, 16)]
                    plsc.addupdate_scatter(
                        rows, [lax.shift_right_logical(idx16, 7),
                               lax.bitwise_and(idx16, 127)], ones16)

            def step(P, sp):
                for a, b in didx_cp(P + 1, 1 - sp):
                    pltpu.async_copy(a, b, semi)
                count(sp)
                for a, b in didx_cp(P + 1, 1 - sp):
                    pltpu.make_async_copy(a, b, semi).wait()

            pltpu.sync_copy(z128_hbm.at[pl.ds(0, K)], rows.at[pl.ds(0, K)])
            for a, b in didx_cp(0, 0):
                pltpu.async_copy(a, b, semi)
                pltpu.make_async_copy(a, b, semi).wait()
            step(0, 0)

            @pl.loop(0, (CPT - 2) // 2)
            def _(j2):
                P = 2 * j2 + 1
                step(P, 1)
                step(P + 1, 0)

            count(1 if CPT % 2 == 0 else 0)

        def run_direction(src_hbm, dst_hbm, xs_out, ea_out, dg_out):
            zero_slab()
            plsc.subcore_barrier()
            run_phase("gather", src_hbm, dst_hbm)
            plsc.subcore_barrier()
            copy_out(xs_out)
            plsc.subcore_barrier()
            zero_slab()
            plsc.subcore_barrier()
            run_phase("seq", src_hbm, dst_hbm)
            run_count(dst_hbm)
            plsc.subcore_barrier()
            copy_out(ea_out)
            pltpu.sync_copy(rows.at[pl.ds(0, K)], dg_out.at[s])

        @pl.when(c == 0)
        def _():
            run_direction(e0_hbm, e1_hbm, xs1_out, ea1_out, dg1_out)

        @pl.when(c == 1)
        def _():
            run_direction(e1_hbm, e0_hbm, xs2_out, ea2_out, dg2_out)

    return sc_kernel(e0, e1, x, ea, z128)


def _combine_body(x_ref, xs1_ref, xs2_ref, ea1_ref, ea2_ref, d1_ref, d2_ref,
                  wn1_ref, we1_ref, wm1_ref, bm1_ref,
                  wn2_ref, we2_ref, wm2_ref, bm2_ref, o_ref):
    f32 = jnp.float32
    D = x_ref.shape[1]

    def mm_t(a, b):
        return lax.dot_general(a, b, (((1,), (1,)), ((), ())),
                               preferred_element_type=f32)

    xb = x_ref[...]
    wn1 = wn1_ref[...]
    wn2 = wn2_ref[...]
    wm1 = wm1_ref[...]
    wm2 = wm2_ref[...]
    d1 = jnp.sum(d1_ref[...], axis=1, keepdims=True)
    d2 = jnp.sum(d2_ref[...], axis=1, keepdims=True)
    agg = (mm_t(mm_t(xs1_ref[...], wn1), wm1[:, 0:D])
           + d1 * mm_t(mm_t(xb, wn1), wm1[:, D:2 * D])
           + mm_t(mm_t(ea1_ref[...], we1_ref[...]), wm1[:, 2 * D:3 * D])
           + d1 * bm1_ref[...]
           + mm_t(mm_t(xs2_ref[...], wn2), wm2[:, 0:D])
           + d2 * mm_t(mm_t(xb, wn2), wm2[:, D:2 * D])
           + mm_t(mm_t(ea2_ref[...], we2_ref[...]), wm2[:, 2 * D:3 * D])
           + d2 * bm2_ref[...])
    o_ref[...] = xb + 0.5 * agg


def _combine(x, xs1, xs2, ea1, ea2, dg1, dg2,
             Wn1, We1, Wm1, bm1, Wn2, We2, Wm2, bm2):
    N, D = x.shape
    NB = 1000
    grid = (N // NB,)
    row_spec = pl.BlockSpec((NB, D), lambda i: (i, 0))
    deg_spec = pl.BlockSpec((NB, NS), lambda i: (i, 0))
    w_spec = pl.BlockSpec((D, D), lambda i: (0, 0))
    wm_spec = pl.BlockSpec((D, 3 * D), lambda i: (0, 0))
    b_spec = pl.BlockSpec((1, D), lambda i: (0, 0))
    return pl.pallas_call(
        _combine_body,
        grid=grid,
        in_specs=[row_spec, row_spec, row_spec, row_spec, row_spec,
                  deg_spec, deg_spec,
                  w_spec, w_spec, wm_spec, b_spec,
                  w_spec, w_spec, wm_spec, b_spec],
        out_specs=row_spec,
        out_shape=jax.ShapeDtypeStruct((N, D), jnp.float32),
    )(x, xs1, xs2, ea1, ea2, dg1, dg2,
      Wn1, We1, Wm1, bm1, Wn2, We2, Wm2, bm2)


def kernel(x, edge_index, edge_attr, memory, batch_id,
           Wn_s2t, We_s2t, Wm_s2t, bm_s2t,
           Wn_t2s, We_t2s, Wm_t2s, bm_t2s):
    N, D = x.shape
    E = edge_index.shape[1]
    assert E % (NS * K * 2) == 0 and N % 1000 == 0 and N % 16 == 0

    z128 = jnp.zeros((N, D), jnp.float32)

    xs1, xs2, ea1, ea2, dg1, dg2 = _sc_segment_sums(
        edge_index[0], edge_index[1], x, edge_attr, z128)
    assert K * D >= N
    dg1 = dg1.reshape(NS, K * D)[:, :N].T
    dg2 = dg2.reshape(NS, K * D)[:, :N].T

    out = _combine(x, xs1, xs2, ea1, ea2, dg1, dg2,
                   Wn_s2t, We_s2t, Wm_s2t, jnp.reshape(bm_s2t, (1, D)),
                   Wn_t2s, We_t2s, Wm_t2s, jnp.reshape(bm_t2s, (1, D)))
    return (out, edge_attr)
